# baseline (speedup 1.0000x reference)
import functools

import jax
import jax.numpy as jnp
from jax import lax
from jax.experimental import pallas as pl
from jax.experimental.pallas import tpu as pltpu
from jax.experimental.pallas import tpu_sc as plsc

B = 4096
T = 20
H = 256
D = 512
N = B * T
K = 128
N2 = N // 2
P = T * T
KP = 128

_info = plsc.get_sparse_core_info()
_NC = _info.num_cores
_NS = _info.num_subcores
_NW = _NC * _NS
_BPW = N2 // _NW
_CH = 320
_NPAIR = _BPW // (2 * _CH)

_RB = 128


def _table2_body(table_ref, w1_ref, b1_ref, w2_ref, b2_ref, out_ref):
    h = jnp.tanh(
        jnp.dot(table_ref[...], w1_ref[...], preferred_element_type=jnp.float32)
        + b1_ref[...]
    )
    t2 = jnp.dot(h, w2_ref[...], preferred_element_type=jnp.float32) + b2_ref[...]
    out_ref[...] = jnp.concatenate(
        [t2, jnp.zeros((K - T, D), jnp.float32)], axis=0
    )


def _padded_table2(table, W1, b1, W2, b2):
    return pl.pallas_call(
        _table2_body,
        out_shape=jax.ShapeDtypeStruct((K, D), jnp.float32),
    )(table, W1, b1.reshape(1, H), W2, b2.reshape(1, D))


_mesh = plsc.VectorSubcoreMesh(core_axis_name="c", subcore_axis_name="s")


@functools.partial(
    pl.kernel,
    mesh=_mesh,
    out_type=jax.ShapeDtypeStruct((N2, KP), jnp.float32),
    scratch_types=[
        pltpu.VMEM((_BPW,), jnp.int32),
        pltpu.VMEM((_CH, KP), jnp.float32),
        pltpu.VMEM((_CH, KP), jnp.float32),
        pltpu.SemaphoreType.DMA,
        pltpu.SemaphoreType.DMA,
    ],
)
def _sc_expand(oh_hbm, idx_hbm, out_hbm, idx_v, buf0, buf1, sem0, sem1):
    wid = lax.axis_index("s") * _NC + lax.axis_index("c")
    base = wid * _BPW
    pltpu.sync_copy(idx_hbm.at[pl.ds(base, _BPW)], idx_v)

    off = (wid * P).astype(jnp.int32)

    def bias(i, carry):
        p = pl.multiple_of(i * 16, 16)
        idx_v[pl.ds(p, 16)] = idx_v[pl.ds(p, 16)] + off
        return carry

    lax.fori_loop(0, _BPW // 16, bias, 0)

    def gather(row_off, buf, sem):
        return pltpu.async_copy(
            oh_hbm.at[idx_v.at[pl.ds(row_off, _CH)]], buf, sem
        )

    def drain(row_off, buf, sem):
        return pltpu.async_copy(
            buf, out_hbm.at[pl.ds(base + row_off, _CH)], sem
        )

    gather(0, buf0, sem0).wait()

    def pair(p, carry):
        off0 = pl.multiple_of(p * 2 * _CH, 2 * _CH)
        cp0 = drain(off0, buf0, sem0)
        g1 = gather(off0 + _CH, buf1, sem1)
        g1.wait()
        cp1 = drain(off0 + _CH, buf1, sem1)
        cp0.wait()

        @pl.when(p < _NPAIR - 1)
        def _():
            gather(off0 + 2 * _CH, buf0, sem0).wait()

        cp1.wait()
        return carry

    lax.fori_loop(0, _NPAIR, pair, 0)


def _select_body(oh_ref, t2_ref, out_ref):
    y = jnp.dot(
        oh_ref[...].astype(jnp.float32),
        t2_ref[...],
        preferred_element_type=jnp.float32,
    )
    out_ref[...] = y.reshape(_RB, T, D)


def _dense_out(oh, t2p):
    return pl.pallas_call(
        _select_body,
        grid=(B // _RB,),
        in_specs=[
            pl.BlockSpec((_RB * T, K), lambda i: (i, 0)),
            pl.BlockSpec((K, D), lambda i: (0, 0)),
        ],
        out_specs=pl.BlockSpec((_RB, T, D), lambda i: (i, 0, 0)),
        out_shape=jax.ShapeDtypeStruct((B, T, D), jnp.float32),
    )(oh, t2p)


def kernel(indices, table, W1, b1, W2, b2):
    t2p = _padded_table2(table, W1, b1, W2, b2)
    eyeb = jnp.eye(T, K, dtype=jnp.bfloat16)
    pair_tab = jnp.concatenate(
        [jnp.repeat(eyeb, T, axis=0), jnp.tile(eyeb, (T, 1))], axis=1
    )
    packed = lax.bitcast_convert_type(pair_tab.reshape(P, KP, 2), jnp.float32)
    oh_table = jnp.tile(packed, (_NW, 1))
    idx = indices.reshape(N2, 2)
    idx2 = idx[:, 0] * T + idx[:, 1]
    oh64 = _sc_expand(oh_table, idx2)
    oh = lax.bitcast_convert_type(oh64, jnp.bfloat16).reshape(N, K)
    return _dense_out(oh, t2p)
